# baseline (speedup 1.0000x reference)
import functools

import jax
import jax.numpy as jnp
from jax import lax
from jax.experimental import pallas as pl
from jax.experimental.pallas import tpu as pltpu
from jax.experimental.pallas import tpu_sc as plsc

NUM_CLASSES = 1000000
EMBED_DIM = 32
BATCH = 16384

_NC = 2
_NS = 16
_NW = _NC * _NS
_B_PER_W = BATCH // _NW
_CHUNK = 128
_NCHUNK = _B_PER_W // _CHUNK

_BS = 8192
_G = -(-NUM_CLASSES // _BS)
_PACKED_ROWS = NUM_CLASSES // 4


def _pack_body(x_ref, o_ref):
    xt = jnp.transpose(x_ref[...], (1, 0))
    xt3 = xt.reshape(_BS // 4, 4, 32)
    for q in range(4):
        o_ref[:, 32 * q:32 * (q + 1)] = xt3[:, q, :]


_tc_pack = pl.pallas_call(
    _pack_body,
    grid=(_G,),
    in_specs=[pl.BlockSpec((32, _BS), lambda k: (0, k))],
    out_specs=pl.BlockSpec((_BS // 4, 128), lambda k: (k, 0)),
    out_shape=jax.ShapeDtypeStruct((_PACKED_ROWS, 128), jnp.float32),
)


def _gather_body(idx_hbm, packed_hbm, out_hbm, idx_v, ridx_v, rows_v,
                 res_v, sem):
    wid = lax.axis_index("s") * _NC + lax.axis_index("c")
    base = wid * _B_PER_W
    for c in range(_NCHUNK):
        pltpu.sync_copy(idx_hbm.at[pl.ds(base + c * _CHUNK, _CHUNK)],
                        idx_v.at[c])
    for c in range(_NCHUNK):
        for s in range(_CHUNK // 16):
            i = idx_v[c, pl.ds(s * 16, 16)]
            ridx_v[c, pl.ds(s * 16, 16)] = lax.shift_right_logical(i, 2)

    copies = [pltpu.async_copy(packed_hbm.at[ridx_v.at[0]], rows_v.at[0],
                               sem)]
    for c in range(_NCHUNK):
        copies[c].wait()
        if c + 1 < _NCHUNK:
            copies.append(pltpu.async_copy(
                packed_hbm.at[ridx_v.at[c + 1]], rows_v.at[(c + 1) % 2],
                sem))

        def gloop(g, _, c=c):
            i16 = idx_v[c, pl.ds(g * 16, 16)]
            for l in range(16):
                i = i16[l]
                q = (i & 3) * 32
                k = g * 16 + l
                res_v[c * _CHUNK + k, pl.ds(0, 16)] = (
                    rows_v[c % 2, k, pl.ds(q, 16)])
                res_v[c * _CHUNK + k, pl.ds(16, 16)] = (
                    rows_v[c % 2, k, pl.ds(q + 16, 16)])
            return ()

        lax.fori_loop(0, _CHUNK // 16, gloop, ())
    pltpu.sync_copy(res_v, out_hbm.at[pl.ds(base, _B_PER_W)])


@jax.jit
def _embed_lookup(class_ids, packed):
    mesh = plsc.VectorSubcoreMesh(core_axis_name="c", subcore_axis_name="s")
    run = functools.partial(
        pl.kernel,
        mesh=mesh,
        out_type=jax.ShapeDtypeStruct((BATCH, EMBED_DIM), jnp.float32),
        scratch_types=[
            pltpu.VMEM((_NCHUNK, _CHUNK), jnp.int32),
            pltpu.VMEM((_NCHUNK, _CHUNK), jnp.int32),
            pltpu.VMEM((2, _CHUNK, 128), jnp.float32),
            pltpu.VMEM((_B_PER_W, EMBED_DIM), jnp.float32),
            pltpu.SemaphoreType.DMA,
        ],
    )(_gather_body)
    return run(class_ids, packed)


def kernel(class_ids, table):
    packed = table.reshape(_PACKED_ROWS, 128)
    return _embed_lookup(class_ids.astype(jnp.int32), packed)


# speedup vs baseline: 1.3843x; 1.3843x over previous
import functools

import jax
import jax.numpy as jnp
from jax import lax
from jax.experimental import pallas as pl
from jax.experimental.pallas import tpu as pltpu
from jax.experimental.pallas import tpu_sc as plsc

NUM_CLASSES = 1000000
EMBED_DIM = 32
BATCH = 16384

_NC = 2
_NS = 16
_NW = _NC * _NS
_B_PER_W = BATCH // _NW
_CHUNK = 128
_NCHUNK = _B_PER_W // _CHUNK

_BS = 16384
_G = -(-NUM_CLASSES // _BS)
_PACKED_ROWS = NUM_CLASSES // 4


def _pack_body(x_ref, o_ref):
    xt = jnp.transpose(x_ref[...], (1, 0))
    xt3 = xt.reshape(_BS // 4, 4, 32)
    for q in range(4):
        o_ref[:, 32 * q:32 * (q + 1)] = xt3[:, q, :]


_tc_pack = pl.pallas_call(
    _pack_body,
    grid=(_G,),
    in_specs=[pl.BlockSpec((32, _BS), lambda k: (0, k))],
    out_specs=pl.BlockSpec((_BS // 4, 128), lambda k: (k, 0)),
    out_shape=jax.ShapeDtypeStruct((_PACKED_ROWS, 128), jnp.float32),
)


def _gather_body(idx_hbm, packed_hbm, out_hbm, idx_v, ridx_v, rows_v,
                 res_v, sem):
    wid = lax.axis_index("s") * _NC + lax.axis_index("c")
    base = wid * _B_PER_W
    for c in range(_NCHUNK):
        pltpu.sync_copy(idx_hbm.at[pl.ds(base + c * _CHUNK, _CHUNK)],
                        idx_v.at[c])
    for c in range(_NCHUNK):
        for s in range(_CHUNK // 16):
            i = idx_v[c, pl.ds(s * 16, 16)]
            ridx_v[c, pl.ds(s * 16, 16)] = lax.shift_right_logical(i, 2)

    copies = [pltpu.async_copy(packed_hbm.at[ridx_v.at[0]], rows_v.at[0],
                               sem)]
    for c in range(_NCHUNK):
        copies[c].wait()
        if c + 1 < _NCHUNK:
            copies.append(pltpu.async_copy(
                packed_hbm.at[ridx_v.at[c + 1]], rows_v.at[(c + 1) % 2],
                sem))

        def gloop(g, _, c=c):
            i16 = idx_v[c, pl.ds(g * 16, 16)]
            for l in range(16):
                i = i16[l]
                q = (i & 3) * 32
                k = g * 16 + l
                res_v[c * _CHUNK + k, pl.ds(0, 16)] = (
                    rows_v[c % 2, k, pl.ds(q, 16)])
                res_v[c * _CHUNK + k, pl.ds(16, 16)] = (
                    rows_v[c % 2, k, pl.ds(q + 16, 16)])
            return ()

        lax.fori_loop(0, _CHUNK // 16, gloop, ())
    pltpu.sync_copy(res_v, out_hbm.at[pl.ds(base, _B_PER_W)])


@jax.jit
def _embed_lookup(class_ids, packed):
    mesh = plsc.VectorSubcoreMesh(core_axis_name="c", subcore_axis_name="s")
    run = functools.partial(
        pl.kernel,
        mesh=mesh,
        out_type=jax.ShapeDtypeStruct((BATCH, EMBED_DIM), jnp.float32),
        scratch_types=[
            pltpu.VMEM((_NCHUNK, _CHUNK), jnp.int32),
            pltpu.VMEM((_NCHUNK, _CHUNK), jnp.int32),
            pltpu.VMEM((2, _CHUNK, 128), jnp.float32),
            pltpu.VMEM((_B_PER_W, EMBED_DIM), jnp.float32),
            pltpu.SemaphoreType.DMA,
        ],
    )(_gather_body)
    return run(class_ids, packed)


def kernel(class_ids, table):
    packed = _tc_pack(table.T)
    return _embed_lookup(class_ids.astype(jnp.int32), packed)
